# Initial kernel scaffold; baseline (speedup 1.0000x reference)
import jax
import jax.numpy as jnp
from jax.experimental import pallas as pl


def kernel(features, alpha):
    raise NotImplementedError("write your pallas kernel here")


# speedup vs baseline: 3.2235x; 3.2235x over previous
import jax
import jax.numpy as jnp
from jax.experimental import pallas as pl

_NUM_ENH = 8


def _body(feat_ref, out_ref):
    f = feat_ref[0]
    s = f.shape[0]
    out_ref[0, :s, :] = f
    total = jnp.sum(f, axis=0, keepdims=True)
    out_ref[0, s:, :] = jnp.broadcast_to(total, (_NUM_ENH, f.shape[1]))


def kernel(features, alpha):
    del alpha
    B, S, D = features.shape
    return pl.pallas_call(
        _body,
        grid=(B,),
        in_specs=[pl.BlockSpec((1, S, D), lambda i: (i, 0, 0))],
        out_specs=pl.BlockSpec((1, S + _NUM_ENH, D), lambda i: (i, 0, 0)),
        out_shape=jax.ShapeDtypeStruct((B, S + _NUM_ENH, D), features.dtype),
    )(features)


# speedup vs baseline: 3.2821x; 1.0182x over previous
import jax
import jax.numpy as jnp
from jax.experimental import pallas as pl
from jax.experimental.pallas import tpu as pltpu

_NUM_ENH = 8


def _body(feat_ref, out_ref):
    f = feat_ref[0]
    s = f.shape[0]
    out_ref[0, :s, :] = f
    total = jnp.sum(f, axis=0, keepdims=True)
    out_ref[0, s:, :] = jnp.broadcast_to(total, (_NUM_ENH, f.shape[1]))


def kernel(features, alpha):
    del alpha
    B, S, D = features.shape
    return pl.pallas_call(
        _body,
        grid=(B,),
        in_specs=[pl.BlockSpec((1, S, D), lambda i: (i, 0, 0))],
        out_specs=pl.BlockSpec((1, S + _NUM_ENH, D), lambda i: (i, 0, 0)),
        out_shape=jax.ShapeDtypeStruct((B, S + _NUM_ENH, D), features.dtype),
        compiler_params=pltpu.CompilerParams(
            dimension_semantics=("parallel",),
        ),
    )(features)


# speedup vs baseline: 5.2659x; 1.6044x over previous
import jax
import jax.numpy as jnp
from jax.experimental import pallas as pl
from jax.experimental.pallas import tpu as pltpu

_NUM_ENH = 8


_BB = 4


def _body(feat_ref, out_ref):
    s = feat_ref.shape[1]
    for b in range(_BB):
        f = feat_ref[b]
        out_ref[b, :s, :] = f
        total = jnp.sum(f, axis=0, keepdims=True)
        out_ref[b, s:, :] = jnp.broadcast_to(total, (_NUM_ENH, f.shape[1]))


def kernel(features, alpha):
    del alpha
    B, S, D = features.shape
    return pl.pallas_call(
        _body,
        grid=(B // _BB,),
        in_specs=[pl.BlockSpec((_BB, S, D), lambda i: (i, 0, 0))],
        out_specs=pl.BlockSpec((_BB, S + _NUM_ENH, D), lambda i: (i, 0, 0)),
        out_shape=jax.ShapeDtypeStruct((B, S + _NUM_ENH, D), features.dtype),
        compiler_params=pltpu.CompilerParams(
            dimension_semantics=("parallel",),
        ),
    )(features)


# speedup vs baseline: 5.3835x; 1.0223x over previous
import jax
import jax.numpy as jnp
from jax.experimental import pallas as pl
from jax.experimental.pallas import tpu as pltpu

_NUM_ENH = 8


_BB = 8


def _body(feat_ref, out_ref):
    s = feat_ref.shape[1]
    for b in range(_BB):
        f = feat_ref[b]
        out_ref[b, :s, :] = f
        total = jnp.sum(f, axis=0, keepdims=True)
        out_ref[b, s:, :] = jnp.broadcast_to(total, (_NUM_ENH, f.shape[1]))


def kernel(features, alpha):
    del alpha
    B, S, D = features.shape
    return pl.pallas_call(
        _body,
        grid=(B // _BB,),
        in_specs=[pl.BlockSpec((_BB, S, D), lambda i: (i, 0, 0))],
        out_specs=pl.BlockSpec((_BB, S + _NUM_ENH, D), lambda i: (i, 0, 0)),
        out_shape=jax.ShapeDtypeStruct((B, S + _NUM_ENH, D), features.dtype),
        compiler_params=pltpu.CompilerParams(
            dimension_semantics=("parallel",),
        ),
    )(features)
